# Initial kernel scaffold; baseline (speedup 1.0000x reference)
import jax
import jax.numpy as jnp
from jax.experimental import pallas as pl


def kernel(idx, targets, table):
    raise NotImplementedError("write your pallas kernel here")


# speedup vs baseline: 1.3688x; 1.3688x over previous
import functools

import jax
import jax.numpy as jnp
from jax import lax
from jax.experimental import pallas as pl
from jax.experimental.pallas import tpu as pltpu
from jax.experimental.pallas import tpu_sc as plsc

_V = 1000
_B = 1024
_T = 200
_N = _B * _T

_NC = 2
_NS = 16
_NW = _NC * _NS
_L = 16
_ROWS_PER_W = _N // _NW
_CHUNK = 64
_NCHUNK = _ROWS_PER_W // _CHUNK


def _lse_body(table_ref, lse_ref):
    x = table_ref[...]
    m = jnp.max(x, axis=1, keepdims=True)
    s = jnp.sum(jnp.exp(x - m), axis=1, keepdims=True)
    lse_ref[...] = m + jnp.log(s)


def _compute_lse(table):
    return pl.pallas_call(
        _lse_body,
        out_shape=jax.ShapeDtypeStruct((_V, 1), jnp.float32),
    )(table)


def _sc_body(table_hbm, idx_hbm, tgt_hbm, lse_hbm, out_hbm, part_hbm,
             idx_v, tgt_v, rows_v, lse_v, acc_v, sem):
    wid = lax.axis_index("s") * _NC + lax.axis_index("c")
    base = wid * _ROWS_PER_W

    pltpu.sync_copy(lse_hbm, lse_v)
    pltpu.sync_copy(idx_hbm.at[pl.ds(base, _ROWS_PER_W)], idx_v)
    pltpu.sync_copy(tgt_hbm.at[pl.ds(base, _ROWS_PER_W)], tgt_v)
    acc_v[...] = jnp.zeros((_L,), jnp.float32)

    def step(g, carry):
        off = g * _CHUNK
        pltpu.async_copy(table_hbm.at[idx_v.at[pl.ds(off, _CHUNK)]],
                         rows_v, sem).wait()
        pltpu.sync_copy(rows_v, out_hbm.at[pl.ds(base + off, _CHUNK)])
        for j in range(_CHUNK // _L):
            iv = idx_v[pl.ds(off + j * _L, _L)]
            tv = tgt_v[pl.ds(off + j * _L, _L)]
            lse_g = plsc.load_gather(lse_v, [iv])
            nloc = lax.iota(jnp.int32, _L) + j * _L
            picked = plsc.load_gather(rows_v, [nloc, tv])
            acc_v[...] = acc_v[...] + (lse_g - picked)
        return carry

    lax.fori_loop(0, _NCHUNK, step, 0)
    pltpu.sync_copy(acc_v, part_hbm.at[wid])


@functools.partial(
    pl.kernel,
    out_type=[
        jax.ShapeDtypeStruct((_N, _V), jnp.float32),
        jax.ShapeDtypeStruct((_NW, _L), jnp.float32),
    ],
    mesh=plsc.VectorSubcoreMesh(core_axis_name="c", subcore_axis_name="s",
                                num_cores=_NC, num_subcores=_NS),
    compiler_params=pltpu.CompilerParams(needs_layout_passes=False,
                                         use_tc_tiling_on_sc=False),
    scratch_types=[
        pltpu.VMEM((_ROWS_PER_W,), jnp.int32),
        pltpu.VMEM((_ROWS_PER_W,), jnp.int32),
        pltpu.VMEM((_CHUNK, _V), jnp.float32),
        pltpu.VMEM((_V,), jnp.float32),
        pltpu.VMEM((_L,), jnp.float32),
        pltpu.SemaphoreType.DMA,
    ],
)
def _sc_kernel(table_hbm, idx_hbm, tgt_hbm, lse_hbm, out_hbm, part_hbm,
               idx_v, tgt_v, rows_v, lse_v, acc_v, sem):
    _sc_body(table_hbm, idx_hbm, tgt_hbm, lse_hbm, out_hbm, part_hbm,
             idx_v, tgt_v, rows_v, lse_v, acc_v, sem)


def kernel(idx, targets, table):
    idx_f = idx.reshape(_N).astype(jnp.int32)
    tgt_f = targets.reshape(_N).astype(jnp.int32)
    lse = _compute_lse(table).reshape(_V)
    logits, part = _sc_kernel(table, idx_f, tgt_f, lse)
    loss = jnp.sum(part) / _N
    return (logits, loss)


# speedup vs baseline: 1.4067x; 1.0277x over previous
import functools

import jax
import jax.numpy as jnp
from jax import lax
from jax.experimental import pallas as pl
from jax.experimental.pallas import tpu as pltpu
from jax.experimental.pallas import tpu_sc as plsc

_V = 1000
_B = 1024
_T = 200
_N = _B * _T

_NC = 2
_NS = 16
_NW = _NC * _NS
_L = 16
_ROWS_PER_W = _N // _NW
_CHUNK = 32
_NCHUNK = _ROWS_PER_W // _CHUNK


def _lse_body(table_ref, lse_ref):
    x = table_ref[...]
    m = jnp.max(x, axis=1, keepdims=True)
    s = jnp.sum(jnp.exp(x - m), axis=1, keepdims=True)
    lse_ref[...] = m + jnp.log(s)


def _compute_lse(table):
    return pl.pallas_call(
        _lse_body,
        out_shape=jax.ShapeDtypeStruct((_V, 1), jnp.float32),
    )(table)


def _sc_body(table_hbm, idx_hbm, tgt_hbm, lse_hbm, out_hbm, part_hbm,
             idx_v, tgt_v, rows0_v, rows1_v, lse_v, acc_v,
             sg0, sg1, so0, so1):
    wid = lax.axis_index("s") * _NC + lax.axis_index("c")
    base = wid * _ROWS_PER_W
    rows = (rows0_v, rows1_v)
    sg = (sg0, sg1)
    so = (so0, so1)

    pltpu.sync_copy(lse_hbm, lse_v)
    pltpu.sync_copy(idx_hbm.at[pl.ds(base, _ROWS_PER_W)], idx_v)
    pltpu.sync_copy(tgt_hbm.at[pl.ds(base, _ROWS_PER_W)], tgt_v)
    acc_v[...] = jnp.zeros((_L,), jnp.float32)

    def gather_chunk(g, b):
        pltpu.async_copy(table_hbm.at[idx_v.at[pl.ds(g * _CHUNK, _CHUNK)]],
                         rows[b], sg[b])

    def picks(g, b):
        off = g * _CHUNK
        for j in range(_CHUNK // _L):
            iv = idx_v[pl.ds(off + j * _L, _L)]
            tv = tgt_v[pl.ds(off + j * _L, _L)]
            lse_g = plsc.load_gather(lse_v, [iv])
            nloc = lax.iota(jnp.int32, _L) + j * _L
            picked = plsc.load_gather(rows[b], [nloc, tv])
            acc_v[...] = acc_v[...] + (lse_g - picked)

    gather_chunk(0, 0)
    gather_chunk(1, 1)

    def step(h, carry):
        for b in range(2):
            g = h * 2 + b
            pltpu.make_async_copy(
                table_hbm.at[idx_v.at[pl.ds(0, _CHUNK)]], rows[b], sg[b]
            ).wait()
            pltpu.async_copy(rows[b], out_hbm.at[pl.ds(base + g * _CHUNK,
                                                       _CHUNK)], so[b])
            picks(g, b)
            @pl.when(g + 2 < _NCHUNK)
            def _():
                pltpu.make_async_copy(
                    rows[b], out_hbm.at[pl.ds(base, _CHUNK)], so[b]
                ).wait()
                gather_chunk(g + 2, b)
        return carry

    lax.fori_loop(0, _NCHUNK // 2, step, 0)
    for b in range(2):
        pltpu.make_async_copy(
            rows[b], out_hbm.at[pl.ds(base, _CHUNK)], so[b]
        ).wait()
    pltpu.sync_copy(acc_v, part_hbm.at[pl.ds(wid * _L, _L)])


@functools.partial(
    pl.kernel,
    out_type=[
        jax.ShapeDtypeStruct((_N, _V), jnp.float32),
        jax.ShapeDtypeStruct((_NW * _L,), jnp.float32),
    ],
    mesh=plsc.VectorSubcoreMesh(core_axis_name="c", subcore_axis_name="s",
                                num_cores=_NC, num_subcores=_NS),
    compiler_params=pltpu.CompilerParams(needs_layout_passes=False,
                                         use_tc_tiling_on_sc=False),
    scratch_types=[
        pltpu.VMEM((_ROWS_PER_W,), jnp.int32),
        pltpu.VMEM((_ROWS_PER_W,), jnp.int32),
        pltpu.VMEM((_CHUNK, _V), jnp.float32),
        pltpu.VMEM((_CHUNK, _V), jnp.float32),
        pltpu.VMEM((_V,), jnp.float32),
        pltpu.VMEM((_L,), jnp.float32),
        pltpu.SemaphoreType.DMA,
        pltpu.SemaphoreType.DMA,
        pltpu.SemaphoreType.DMA,
        pltpu.SemaphoreType.DMA,
    ],
)
def _sc_kernel(table_hbm, idx_hbm, tgt_hbm, lse_hbm, out_hbm, part_hbm,
               idx_v, tgt_v, rows0_v, rows1_v, lse_v, acc_v,
               sg0, sg1, so0, so1):
    _sc_body(table_hbm, idx_hbm, tgt_hbm, lse_hbm, out_hbm, part_hbm,
             idx_v, tgt_v, rows0_v, rows1_v, lse_v, acc_v,
             sg0, sg1, so0, so1)


def kernel(idx, targets, table):
    idx_f = idx.reshape(_N).astype(jnp.int32)
    tgt_f = targets.reshape(_N).astype(jnp.int32)
    lse = _compute_lse(table).reshape(_V)
    logits, part = _sc_kernel(table, idx_f, tgt_f, lse)
    loss = jnp.sum(part) / _N
    return (logits, loss)


# speedup vs baseline: 1.4067x; 1.0000x over previous
import functools

import jax
import jax.numpy as jnp
from jax import lax
from jax.experimental import pallas as pl
from jax.experimental.pallas import tpu as pltpu
from jax.experimental.pallas import tpu_sc as plsc

_V = 1000
_S = 8
_P = 128
_B = 1024
_T = 200
_N = _B * _T

_NC = 2
_NS = 16
_NW = _NC * _NS
_L = 16
_ROWS_PER_W = _N // _NW
_CHUNK = 16
_CP = _CHUNK * _S
_NCHUNK = _ROWS_PER_W // _CHUNK


def _lse_body(table_ref, lse_ref):
    x = table_ref[...]
    m = jnp.max(x, axis=1, keepdims=True)
    s = jnp.sum(jnp.exp(x - m), axis=1, keepdims=True)
    lse_ref[...] = m + jnp.log(s)


def _compute_lse(table):
    return pl.pallas_call(
        _lse_body,
        out_shape=jax.ShapeDtypeStruct((_V, 1), jnp.float32),
    )(table)


def _sc_body(table_hbm, idx_hbm, tgt_hbm, lse_hbm, out_hbm, part_hbm,
             idx_v, tgt_v, idx8_v, rows0_v, rows1_v, lse_v, acc_v,
             sg0, sg1, so0, so1):
    wid = lax.axis_index("s") * _NC + lax.axis_index("c")
    base = wid * _ROWS_PER_W
    rows = (rows0_v, rows1_v)
    sg = (sg0, sg1)
    so = (so0, so1)

    pltpu.sync_copy(lse_hbm, lse_v)
    pltpu.sync_copy(idx_hbm.at[pl.ds(base, _ROWS_PER_W)], idx_v)
    pltpu.sync_copy(tgt_hbm.at[pl.ds(base, _ROWS_PER_W)], tgt_v)
    acc_v[...] = jnp.zeros((_L,), jnp.float32)

    lanes = lax.iota(jnp.int32, _L)
    cvec = lanes & 7

    def build(j, carry):
        k = j * _L
        nvec = (lanes + k) >> 3
        iv = plsc.load_gather(idx_v, [nvec])
        idx8_v[pl.ds(k, _L)] = iv * _S + cvec
        return carry

    lax.fori_loop(0, _ROWS_PER_W * _S // _L, build, 0)

    def gather_chunk(g, b):
        pltpu.async_copy(table_hbm.at[idx8_v.at[pl.ds(g * _CP, _CP)]],
                         rows[b], sg[b])

    def picks(g, b):
        off = g * _CHUNK
        iv = idx_v[pl.ds(off, _L)]
        tv = tgt_v[pl.ds(off, _L)]
        lse_g = plsc.load_gather(lse_v, [iv])
        qi = tv // _P
        rm = tv - qi * _P
        picked = plsc.load_gather(rows[b], [lanes * _S + qi, rm])
        acc_v[...] = acc_v[...] + (lse_g - picked)

    gather_chunk(0, 0)
    gather_chunk(1, 1)

    def step(h, carry):
        for b in range(2):
            g = h * 2 + b
            pltpu.make_async_copy(
                table_hbm.at[idx8_v.at[pl.ds(0, _CP)]], rows[b], sg[b]
            ).wait()
            pltpu.async_copy(rows[b],
                             out_hbm.at[pl.ds((base + g * _CHUNK) * _S, _CP)],
                             so[b])
            picks(g, b)
            @pl.when(g + 2 < _NCHUNK)
            def _():
                pltpu.make_async_copy(
                    rows[b], out_hbm.at[pl.ds(base * _S, _CP)], so[b]
                ).wait()
                gather_chunk(g + 2, b)
        return carry

    lax.fori_loop(0, _NCHUNK // 2, step, 0)
    for b in range(2):
        pltpu.make_async_copy(
            rows[b], out_hbm.at[pl.ds(base * _S, _CP)], so[b]
        ).wait()
    pltpu.sync_copy(acc_v, part_hbm.at[pl.ds(wid * _L, _L)])


@functools.partial(
    pl.kernel,
    out_type=[
        jax.ShapeDtypeStruct((_N * _S, _P), jnp.float32),
        jax.ShapeDtypeStruct((_NW * _L,), jnp.float32),
    ],
    mesh=plsc.VectorSubcoreMesh(core_axis_name="c", subcore_axis_name="s",
                                num_cores=_NC, num_subcores=_NS),
    compiler_params=pltpu.CompilerParams(needs_layout_passes=False,
                                         use_tc_tiling_on_sc=False),
    scratch_types=[
        pltpu.VMEM((_ROWS_PER_W,), jnp.int32),
        pltpu.VMEM((_ROWS_PER_W,), jnp.int32),
        pltpu.VMEM((_ROWS_PER_W * _S,), jnp.int32),
        pltpu.VMEM((_CP, _P), jnp.float32),
        pltpu.VMEM((_CP, _P), jnp.float32),
        pltpu.VMEM((_V,), jnp.float32),
        pltpu.VMEM((_L,), jnp.float32),
        pltpu.SemaphoreType.DMA,
        pltpu.SemaphoreType.DMA,
        pltpu.SemaphoreType.DMA,
        pltpu.SemaphoreType.DMA,
    ],
)
def _sc_kernel(table_hbm, idx_hbm, tgt_hbm, lse_hbm, out_hbm, part_hbm,
               idx_v, tgt_v, idx8_v, rows0_v, rows1_v, lse_v, acc_v,
               sg0, sg1, so0, so1):
    _sc_body(table_hbm, idx_hbm, tgt_hbm, lse_hbm, out_hbm, part_hbm,
             idx_v, tgt_v, idx8_v, rows0_v, rows1_v, lse_v, acc_v,
             sg0, sg1, so0, so1)


def kernel(idx, targets, table):
    idx_f = idx.reshape(_N).astype(jnp.int32)
    tgt_f = targets.reshape(_N).astype(jnp.int32)
    table8 = jnp.pad(table, ((0, 0), (0, _S * _P - _V))).reshape(_V * _S, _P)
    lse = _compute_lse(table).reshape(_V)
    out8, part = _sc_kernel(table8, idx_f, tgt_f, lse)
    logits = out8.reshape(_N, _S * _P)[:, :_V]
    loss = jnp.sum(part) / _N
    return (logits, loss)
